# Initial kernel scaffold; baseline (speedup 1.0000x reference)
import jax
import jax.numpy as jnp
from jax.experimental import pallas as pl


def kernel(x, edge_index, W, b):
    raise NotImplementedError("write your pallas kernel here")


# speedup vs baseline: 4.3195x; 4.3195x over previous
import functools

import jax
import jax.numpy as jnp
from jax import lax
from jax.experimental import pallas as pl
from jax.experimental.pallas import tpu as pltpu
from jax.experimental.pallas import tpu_sc as plsc

_NC = 2
_NS = 16
_LANES = 16
_CHUNK = 128
_ZROWS = 40


def _tc_linear_body(a0_ref, a1_ref, w0_ref, w1_ref, b_ref, o_ref):
    h = jnp.dot(a0_ref[0], w0_ref[...], preferred_element_type=jnp.float32)
    h = h + jnp.dot(a1_ref[0], w1_ref[...], preferred_element_type=jnp.float32)
    o_ref[...] = jnp.tanh(h + b_ref[...])


def _make_sc_aggregate(n, dh, kc, rows_per_tile):
    rows_sh = _NS * rows_per_tile
    mesh = plsc.VectorSubcoreMesh(core_axis_name="c", subcore_axis_name="s")

    @functools.partial(
        pl.kernel,
        out_type=jax.ShapeDtypeStruct((_NC, rows_sh, dh), jnp.float32),
        mesh=mesh,
        scratch_types=[
            pltpu.VMEM((kc, _CHUNK), jnp.int32),
            pltpu.VMEM((kc, _CHUNK), jnp.int32),
            pltpu.VMEM((_CHUNK, dh), jnp.float32),
            pltpu.VMEM((_ZROWS, dh), jnp.float32),
            pltpu.VMEM_SHARED((rows_sh, dh), jnp.float32),
            pltpu.SemaphoreType.DMA,
        ],
    )
    def agg_kernel(x_hbm, src_hbm, dst_hbm, out_hbm, sidx, didx, rows, zrow, acc, sem):
        c = lax.axis_index("c")
        s = lax.axis_index("s")

        zv = jnp.zeros((_LANES,), jnp.float32)
        for i in range(_ZROWS):
            for j in range(dh // _LANES):
                zrow[i, pl.ds(j * _LANES, _LANES)] = zv
        zbase = s * rows_per_tile
        for t in range(rows_per_tile // _ZROWS):
            pltpu.sync_copy(zrow, acc.at[pl.ds(zbase + t * _ZROWS, _ZROWS)])

        pltpu.sync_copy(src_hbm.at[c, s], sidx)
        pltpu.sync_copy(dst_hbm.at[s], didx)

        plsc.subcore_barrier()

        def chunk_body(k, carry):
            pltpu.async_copy(x_hbm.at[sidx.at[k]], rows, sem).wait()
            pltpu.sync_copy(rows, acc.at[didx.at[k]], add=True)
            return carry

        lax.fori_loop(0, kc, chunk_body, 0)

        plsc.subcore_barrier()

        rbase = s * rows_per_tile
        off = 0
        while off < rows_per_tile:
            w = min(_CHUNK, rows_per_tile - off)
            pltpu.sync_copy(acc.at[pl.ds(rbase + off, w)], rows.at[pl.ds(0, w)])
            pltpu.sync_copy(rows.at[pl.ds(0, w)], out_hbm.at[c, pl.ds(rbase + off, w)])
            off += w

    return agg_kernel


def kernel(x, edge_index, W, b):
    n, d = x.shape
    e = edge_index.shape[1]
    dh = d // 2

    src = edge_index[0].astype(jnp.int32)
    dst = edge_index[1].astype(jnp.int32)

    epb = _NS * _CHUNK
    e_pad = -(-e // epb) * epb
    pad = e_pad - e
    if pad:
        src = jnp.concatenate([src, jnp.zeros((pad,), jnp.int32)])
        dst = jnp.concatenate([dst, jnp.full((pad,), n, jnp.int32)])
    kc = e_pad // epb

    xflat = x.reshape(n * 2, dh)
    src2 = jnp.stack([2 * src, 2 * src + 1]).reshape(_NC, _NS, kc, _CHUNK)
    dst3 = dst.reshape(_NS, kc, _CHUNK)

    rows_per_tile = -(-(-(-(n + 1) // _NS)) // _ZROWS) * _ZROWS

    agg3 = _make_sc_aggregate(n, dh, kc, rows_per_tile)(xflat, src2, dst3)

    rblk = 1000
    tc = pl.pallas_call(
        _tc_linear_body,
        grid=(n // rblk,),
        in_specs=[
            pl.BlockSpec((1, rblk, dh), lambda i: (0, i, 0)),
            pl.BlockSpec((1, rblk, dh), lambda i: (1, i, 0)),
            pl.BlockSpec((dh, d), lambda i: (0, 0)),
            pl.BlockSpec((dh, d), lambda i: (0, 0)),
            pl.BlockSpec((1, d), lambda i: (0, 0)),
        ],
        out_specs=pl.BlockSpec((rblk, d), lambda i: (i, 0)),
        out_shape=jax.ShapeDtypeStruct((n, d), jnp.float32),
    )
    wt = W.T
    return tc(agg3, agg3, wt[:dh], wt[dh:], b.reshape(1, d))
